# baseline (speedup 1.0000x reference)
import math

import jax
import jax.numpy as jnp
import numpy as np
from jax import lax
from jax.experimental import pallas as pl
from jax.experimental.pallas import tpu as pltpu
from jax.experimental.pallas import tpu_sc as plsc

D = 64
SEQ = 8192
NW = 32
C = 256


def _pos_encoding(seq_len: int, dim: int) -> np.ndarray:
    positions = np.arange(seq_len, dtype=np.float32)[:, None]
    div = np.exp(np.arange(0, dim, 2, dtype=np.float32) * -(math.log(10000.0) / dim))
    pe = np.zeros((seq_len, dim), dtype=np.float32)
    pe[:, 0::2] = np.sin(positions * div)
    pe[:, 1::2] = np.cos(positions * div)
    return pe


_POS = _pos_encoding(SEQ, D)


def _sc_embed(ids_hbm, pos_hbm, table_hbm, out_hbm, idx_v, rows_v, pos_v, gsem):
    bpw = ids_hbm.shape[0] // NW
    nchunk = bpw // C
    wid = lax.axis_index("s") * 2 + lax.axis_index("c")
    base = pl.multiple_of(wid * bpw, bpw)
    pos0 = pl.multiple_of(lax.rem(base, SEQ), C)
    pltpu.sync_copy(ids_hbm.at[pl.ds(base, bpw)], idx_v)

    def chunk_body(g, _):
        off = pl.multiple_of(g * C, C)
        pltpu.async_copy(table_hbm.at[idx_v.at[pl.ds(off, C)]], rows_v, gsem).wait()
        pltpu.sync_copy(pos_hbm.at[pl.ds(pos0 + off, C)], pos_v)

        def add_body(r, _):
            for j in range(D // 16):
                sl = pl.ds(j * 16, 16)
                rows_v[r, sl] = rows_v[r, sl] + pos_v[r, sl]
            return 0

        lax.fori_loop(0, C, add_body, 0, unroll=4)
        pltpu.sync_copy(rows_v, out_hbm.at[pl.ds(base + off, C)])
        return 0

    lax.fori_loop(0, nchunk, chunk_body, 0)


def kernel(token_ids, token_embedding_weight):
    b, s = token_ids.shape
    ids_flat = token_ids.reshape(-1).astype(jnp.int32)
    pos = jnp.asarray(_POS)

    run = pl.kernel(
        _sc_embed,
        out_type=jax.ShapeDtypeStruct((b * s, D), jnp.float32),
        mesh=plsc.VectorSubcoreMesh(core_axis_name="c", subcore_axis_name="s"),
        scratch_types=[
            pltpu.VMEM((b * s // NW,), jnp.int32),
            pltpu.VMEM((C, D), jnp.float32),
            pltpu.VMEM((C, D), jnp.float32),
            pltpu.SemaphoreType.DMA,
        ],
        compiler_params=pltpu.CompilerParams(use_tc_tiling_on_sc=False),
    )
    out = run(ids_flat, pos, token_embedding_weight)
    return out.reshape(b, s, D)


# speedup vs baseline: 2.1941x; 2.1941x over previous
import math

import jax
import jax.numpy as jnp
import numpy as np
from jax import lax
from jax.experimental import pallas as pl
from jax.experimental.pallas import tpu as pltpu
from jax.experimental.pallas import tpu_sc as plsc

D = 64
SEQ = 8192
NW = 32
BPW = 1024
CG = 32
CO = 128
NSUB = BPW // CG
SUBS_PER_OUT = CO // CG
NCO = BPW // CO
V3 = 125000


def _pos_encoding(seq_len: int, dim: int) -> np.ndarray:
    positions = np.arange(seq_len, dtype=np.float32)[:, None]
    div = np.exp(np.arange(0, dim, 2, dtype=np.float32) * -(math.log(10000.0) / dim))
    pe = np.zeros((seq_len, dim), dtype=np.float32)
    pe[:, 0::2] = np.sin(positions * div)
    pe[:, 1::2] = np.cos(positions * div)
    return pe


_POS_T = np.ascontiguousarray(_pos_encoding(SEQ, D).T)


def _sc_embed(ids_hbm, post_hbm, table_hbm, out_hbm,
              ids_v, rows_v, outt_v, gsem, psem, osem):
    wid = lax.axis_index("s") * 2 + lax.axis_index("c")
    base = pl.multiple_of(wid * BPW, BPW)
    b = lax.div(base, SEQ)
    s0 = pl.multiple_of(lax.rem(base, SEQ), BPW)
    lane = lax.iota(jnp.int32, 16)

    pltpu.sync_copy(ids_hbm.at[pl.ds(base, BPW)], ids_v)

    def issue_sub(gs, buf):
        goff = gs * CG

        def issue_grp(m):
            k16 = lax.shift_right_logical(ids_v[pl.ds(goff + m * 16, 16)], 3)
            ks = [jnp.max(jnp.where(lane == l, k16, 0)) for l in range(16)]
            for l in range(16):
                pltpu.async_copy(table_hbm.at[ks[l]],
                                 rows_v.at[buf * CG + m * 16 + l], gsem)

        issue_grp(0)
        issue_grp(1)

    def issue_pos(co, cobuf):
        pltpu.async_copy(post_hbm.at[:, pl.ds(s0 + co * CO, CO)],
                         outt_v.at[cobuf], psem)

    issue_sub(0, 0)
    issue_pos(0, 0)

    def sub_body(gs, _):
        buf = lax.rem(gs, 2)
        sub_in_out = lax.rem(gs, SUBS_PER_OUT)
        co = lax.div(gs, SUBS_PER_OUT)
        cobuf = lax.rem(co, 2)

        @pl.when(gs + 1 < NSUB)
        def _():
            issue_sub(gs + 1, 1 - buf)

        @pl.when(sub_in_out == 0)
        def _():
            pltpu.make_async_copy(post_hbm.at[:, pl.ds(0, CO)],
                                  outt_v.at[cobuf], psem).wait()

        pltpu.make_async_copy(table_hbm.at[pl.ds(0, CG)],
                              rows_v.at[pl.ds(0, CG)], gsem).wait()

        goff = gs * CG

        def extract_grp(m):
            row0 = m * 16
            s16 = lax.bitwise_and(ids_v[pl.ds(goff + row0, 16)], 7)
            rix = lane + (buf * CG + row0)
            sl = pl.ds(sub_in_out * CG + row0, 16)
            for c in range(D):
                cvec = jnp.full((16,), c, jnp.int32)
                v = plsc.load_gather(rows_v, [rix, s16, cvec])
                plsc.addupdate(outt_v.at[cobuf, c, sl], v)

        extract_grp(0)
        extract_grp(1)

        @pl.when(sub_in_out == SUBS_PER_OUT - 1)
        def _():
            pltpu.async_copy(outt_v.at[cobuf],
                             out_hbm.at[b, :, pl.ds(s0 + co * CO, CO)], osem)

            @pl.when(co >= 1)
            def _():
                pltpu.make_async_copy(outt_v.at[0],
                                      out_hbm.at[0, :, pl.ds(0, CO)],
                                      osem).wait()

            @pl.when(co + 1 < NCO)
            def _():
                issue_pos(co + 1, 1 - cobuf)

        return 0

    lax.fori_loop(0, NSUB, sub_body, 0)
    pltpu.make_async_copy(outt_v.at[0], out_hbm.at[0, :, pl.ds(0, CO)], osem).wait()


def kernel(token_ids, token_embedding_weight):
    nb, ns = token_ids.shape
    ids_flat = token_ids.reshape(-1).astype(jnp.int32)
    post = jnp.asarray(_POS_T)
    table3 = token_embedding_weight.reshape(V3, 8, D)

    run = pl.kernel(
        _sc_embed,
        out_type=jax.ShapeDtypeStruct((nb, D, ns), jnp.float32),
        mesh=plsc.VectorSubcoreMesh(core_axis_name="c", subcore_axis_name="s"),
        scratch_types=[
            pltpu.VMEM((BPW,), jnp.int32),
            pltpu.VMEM((2 * CG, 8, D), jnp.float32),
            pltpu.VMEM((2, D, CO), jnp.float32),
            pltpu.SemaphoreType.DMA,
            pltpu.SemaphoreType.DMA,
            pltpu.SemaphoreType.DMA,
        ],
        compiler_params=pltpu.CompilerParams(needs_layout_passes=False),
    )
    out = run(ids_flat, post, table3)
    return out.transpose(0, 2, 1)


# speedup vs baseline: 2.2037x; 1.0044x over previous
import math

import jax
import jax.numpy as jnp
import numpy as np
from jax import lax
from jax.experimental import pallas as pl
from jax.experimental.pallas import tpu as pltpu
from jax.experimental.pallas import tpu_sc as plsc

D = 64
SEQ = 8192
NW = 32
BPW = 1024
CG = 32
CO = 128
NSUB = BPW // CG
SUBS_PER_OUT = CO // CG
NCO = BPW // CO
V3 = 125000


def _pos_encoding(seq_len: int, dim: int) -> np.ndarray:
    positions = np.arange(seq_len, dtype=np.float32)[:, None]
    div = np.exp(np.arange(0, dim, 2, dtype=np.float32) * -(math.log(10000.0) / dim))
    pe = np.zeros((seq_len, dim), dtype=np.float32)
    pe[:, 0::2] = np.sin(positions * div)
    pe[:, 1::2] = np.cos(positions * div)
    return pe


_POS_T = np.ascontiguousarray(_pos_encoding(SEQ, D).T)


def _sc_embed(ids_hbm, post_hbm, table_hbm, out_hbm,
              ids_v, rows_v, outt_v, gsem, psem, osem):
    wid = lax.axis_index("s") * 2 + lax.axis_index("c")
    base = pl.multiple_of(wid * BPW, BPW)
    b = lax.div(base, SEQ)
    s0 = pl.multiple_of(lax.rem(base, SEQ), BPW)
    lane = lax.iota(jnp.int32, 16)

    pltpu.sync_copy(ids_hbm.at[pl.ds(base, BPW)], ids_v)

    def issue_sub(gs, buf):
        goff = gs * CG

        def issue_grp(m):
            k16 = lax.shift_right_logical(ids_v[pl.ds(goff + m * 16, 16)], 3)
            ks = [jnp.max(jnp.where(lane == l, k16, 0)) for l in range(16)]
            for l in range(16):
                pltpu.async_copy(table_hbm.at[ks[l]],
                                 rows_v.at[buf * CG + m * 16 + l], gsem)

        issue_grp(0)
        issue_grp(1)

    def issue_pos(co, cobuf):
        pltpu.async_copy(post_hbm.at[:, pl.ds(s0 + co * CO, CO)],
                         outt_v.at[cobuf], psem)

    issue_sub(0, 0)
    issue_sub(1, 1)
    issue_pos(0, 0)

    def sub_body(gs, _):
        buf = lax.rem(gs, 3)
        sub_in_out = lax.rem(gs, SUBS_PER_OUT)
        co = lax.div(gs, SUBS_PER_OUT)
        cobuf = lax.rem(co, 2)

        @pl.when(gs + 2 < NSUB)
        def _():
            issue_sub(gs + 2, lax.rem(gs + 2, 3))

        @pl.when(sub_in_out == 0)
        def _():
            pltpu.make_async_copy(post_hbm.at[:, pl.ds(0, CO)],
                                  outt_v.at[cobuf], psem).wait()

        pltpu.make_async_copy(table_hbm.at[pl.ds(0, CG)],
                              rows_v.at[pl.ds(0, CG)], gsem).wait()

        goff = gs * CG

        def extract_grp(m):
            row0 = m * 16
            s16 = lax.bitwise_and(ids_v[pl.ds(goff + row0, 16)], 7)
            rix = lane + (buf * CG + row0)
            sl = pl.ds(sub_in_out * CG + row0, 16)
            for c in range(D):
                cvec = jnp.full((16,), c, jnp.int32)
                v = plsc.load_gather(rows_v, [rix, s16, cvec])
                plsc.addupdate(outt_v.at[cobuf, c, sl], v)

        extract_grp(0)
        extract_grp(1)

        @pl.when(sub_in_out == SUBS_PER_OUT - 1)
        def _():
            pltpu.async_copy(outt_v.at[cobuf],
                             out_hbm.at[b, :, pl.ds(s0 + co * CO, CO)], osem)

            @pl.when(co >= 1)
            def _():
                pltpu.make_async_copy(outt_v.at[0],
                                      out_hbm.at[0, :, pl.ds(0, CO)],
                                      osem).wait()

            @pl.when(co + 1 < NCO)
            def _():
                issue_pos(co + 1, 1 - cobuf)

        return 0

    lax.fori_loop(0, NSUB, sub_body, 0)
    pltpu.make_async_copy(outt_v.at[0], out_hbm.at[0, :, pl.ds(0, CO)], osem).wait()


def kernel(token_ids, token_embedding_weight):
    nb, ns = token_ids.shape
    ids_flat = token_ids.reshape(-1).astype(jnp.int32)
    post = jnp.asarray(_POS_T)
    table3 = token_embedding_weight.reshape(V3, 8, D)

    run = pl.kernel(
        _sc_embed,
        out_type=jax.ShapeDtypeStruct((nb, D, ns), jnp.float32),
        mesh=plsc.VectorSubcoreMesh(core_axis_name="c", subcore_axis_name="s"),
        scratch_types=[
            pltpu.VMEM((BPW,), jnp.int32),
            pltpu.VMEM((3 * CG, 8, D), jnp.float32),
            pltpu.VMEM((2, D, CO), jnp.float32),
            pltpu.SemaphoreType.DMA,
            pltpu.SemaphoreType.DMA,
            pltpu.SemaphoreType.DMA,
        ],
        compiler_params=pltpu.CompilerParams(needs_layout_passes=False),
    )
    out = run(ids_flat, post, table3)
    return out.transpose(0, 2, 1)
